# Initial kernel scaffold; baseline (speedup 1.0000x reference)
import jax
import jax.numpy as jnp
from jax.experimental import pallas as pl


def kernel(x, indices, values):
    raise NotImplementedError("write your pallas kernel here")


# speedup vs baseline: 2.2560x; 2.2560x over previous
import jax
import jax.numpy as jnp
from jax import lax
from jax.experimental import pallas as pl
from jax.experimental.pallas import tpu as pltpu
from jax.experimental.pallas import tpu_sc as plsc

M, N = 4096, 4096
B, C = 2, 256
NC, NS, L = 2, 16, 16
KE = 2048


def _spmm_body(nch):
    def body(xt_hbm, cols_hbm, rows_hbm, vals_hbm, out_hbm,
             acc_v, buf_v, colsc_v, idx_v, rows_v, vals_v, sem):
        c = lax.axis_index("c")
        s = lax.axis_index("s")
        base = (c * NS + s) * N
        iota = lax.broadcasted_iota(jnp.int32, (L,), 0)

        def zero(i, carry):
            acc_v[i, :] = jnp.zeros((L,), jnp.float32)
            return carry
        lax.fori_loop(0, M, zero, 0)

        def chunk(ch, carry):
            off = ch * KE
            pltpu.sync_copy(cols_hbm.at[pl.ds(off, KE)], colsc_v)
            pltpu.sync_copy(rows_hbm.at[pl.ds(off, KE)], rows_v)
            pltpu.sync_copy(vals_hbm.at[pl.ds(off, KE)], vals_v)

            def mkidx(q, carry2):
                sl = pl.ds(q * L, L)
                idx_v[sl] = colsc_v[sl] + base
                return carry2
            lax.fori_loop(0, KE // L, mkidx, 0)

            pltpu.async_copy(xt_hbm.at[idx_v], buf_v, sem).wait()

            def accum(g, carry2):
                rv = rows_v[pl.ds(g * L, L)]
                vv = vals_v[pl.ds(g * L, L)]
                for i in range(L):
                    row = buf_v[g * L + i, :] * vv[i]
                    plsc.addupdate_scatter(
                        acc_v, [jnp.full((L,), rv[i], jnp.int32), iota], row)
                return carry2
            lax.fori_loop(0, KE // L, accum, 0)
            return carry
        lax.fori_loop(0, nch, chunk, 0)

        pltpu.sync_copy(acc_v, out_hbm.at[c, s])

    return body


def kernel(x, indices, values):
    nnz = values.shape[0]
    nch = -(-nnz // KE)
    pad = nch * KE - nnz

    rows = jnp.concatenate([indices[0], jnp.zeros((pad,), jnp.int32)])
    cols = jnp.concatenate([indices[1], jnp.zeros((pad,), jnp.int32)])
    vals = jnp.concatenate([values.astype(jnp.float32),
                            jnp.zeros((pad,), jnp.float32)])
    xt = (x.astype(jnp.float32)
          .reshape(B, N, NS, L).transpose(0, 2, 1, 3).reshape(B * NS * N, L))

    mesh = plsc.VectorSubcoreMesh(core_axis_name="c", subcore_axis_name="s",
                                  num_cores=NC, num_subcores=NS)
    run = pl.kernel(
        _spmm_body(nch),
        out_type=jax.ShapeDtypeStruct((B, NS, M, L), jnp.float32),
        mesh=mesh,
        compiler_params=pltpu.CompilerParams(needs_layout_passes=False,
                                             use_tc_tiling_on_sc=False),
        scratch_types=[
            pltpu.VMEM((M, L), jnp.float32),
            pltpu.VMEM((KE, L), jnp.float32),
            pltpu.VMEM((KE,), jnp.int32),
            pltpu.VMEM((KE,), jnp.int32),
            pltpu.VMEM((KE,), jnp.int32),
            pltpu.VMEM((KE,), jnp.float32),
            pltpu.SemaphoreType.DMA,
        ],
    )
    out = run(xt, cols, rows, vals)
    return out.transpose(0, 2, 1, 3).reshape(B, M, C)


# speedup vs baseline: 2.2824x; 1.0117x over previous
import jax
import jax.numpy as jnp
from jax import lax
from jax.experimental import pallas as pl
from jax.experimental.pallas import tpu as pltpu
from jax.experimental.pallas import tpu_sc as plsc

M, N = 4096, 4096
B, C = 2, 256
NC, NS, L = 2, 16, 16
KE = 2048


def _spmm_body(nch):
    def body(xt_hbm, cols_hbm, rows_hbm, vals_hbm, out_hbm,
             acc_v, buf_v, colsc_v, idx_v, rows_v, vals_v, sem):
        c = lax.axis_index("c")
        s = lax.axis_index("s")
        base = (c * NS + s) * N
        iota = lax.broadcasted_iota(jnp.int32, (L,), 0)
        lane_consts = [jnp.full((L,), i, jnp.int32) for i in range(L)]

        def zero(i, carry):
            for j in range(L):
                acc_v[pl.ds((i * L + j) * L, L)] = jnp.zeros((L,), jnp.float32)
            return carry
        lax.fori_loop(0, M // L, zero, 0)

        def chunk(ch, carry):
            off = ch * KE
            pltpu.sync_copy(cols_hbm.at[pl.ds(off, KE)], colsc_v)
            pltpu.sync_copy(rows_hbm.at[pl.ds(off, KE)], rows_v)
            pltpu.sync_copy(vals_hbm.at[pl.ds(off, KE)], vals_v)

            def mkidx(q, carry2):
                sl = pl.ds(q * L, L)
                idx_v[sl] = colsc_v[sl] + base
                rows_v[sl] = rows_v[sl] * L
                return carry2
            lax.fori_loop(0, KE // L, mkidx, 0)

            pltpu.async_copy(xt_hbm.at[idx_v], buf_v, sem).wait()

            def accum(g, carry2):
                rv16 = rows_v[pl.ds(g * L, L)]
                vv = vals_v[pl.ds(g * L, L)]
                for i in range(L):
                    row = buf_v[g * L + i, :] * vv[lane_consts[i]]
                    idx = rv16[lane_consts[i]] + iota
                    plsc.addupdate_scatter(acc_v, [idx], row)
                return carry2
            lax.fori_loop(0, KE // L, accum, 0)
            return carry
        lax.fori_loop(0, nch, chunk, 0)

        pltpu.sync_copy(acc_v, out_hbm.at[c, s])

    return body


def kernel(x, indices, values):
    nnz = values.shape[0]
    nch = -(-nnz // KE)
    pad = nch * KE - nnz

    rows = jnp.concatenate([indices[0], jnp.zeros((pad,), jnp.int32)])
    cols = jnp.concatenate([indices[1], jnp.zeros((pad,), jnp.int32)])
    vals = jnp.concatenate([values.astype(jnp.float32),
                            jnp.zeros((pad,), jnp.float32)])
    xt = (x.astype(jnp.float32)
          .reshape(B, N, NS, L).transpose(0, 2, 1, 3).reshape(B * NS * N, L))

    mesh = plsc.VectorSubcoreMesh(core_axis_name="c", subcore_axis_name="s",
                                  num_cores=NC, num_subcores=NS)
    run = pl.kernel(
        _spmm_body(nch),
        out_type=jax.ShapeDtypeStruct((B, NS, M * L), jnp.float32),
        mesh=mesh,
        compiler_params=pltpu.CompilerParams(needs_layout_passes=False,
                                             use_tc_tiling_on_sc=False),
        scratch_types=[
            pltpu.VMEM((M * L,), jnp.float32),
            pltpu.VMEM((KE, L), jnp.float32),
            pltpu.VMEM((KE,), jnp.int32),
            pltpu.VMEM((KE,), jnp.int32),
            pltpu.VMEM((KE,), jnp.int32),
            pltpu.VMEM((KE,), jnp.float32),
            pltpu.SemaphoreType.DMA,
        ],
    )
    out = run(xt, cols, rows, vals)
    return out.reshape(B, NS, M, L).transpose(0, 2, 1, 3).reshape(B, M, C)
